# baseline (speedup 1.0000x reference)
import functools

import jax
import jax.numpy as jnp
from jax import lax
from jax.experimental import pallas as pl
from jax.experimental.pallas import tpu as pltpu
from jax.experimental.pallas import tpu_sc as plsc

N_TOKEN = 1000000
D_PROJ = 128
CUT1 = 20000
CUT2 = 200000
SCALE = float(D_PROJ) ** 0.5

N = 16384 * 20
NC, NS = 2, 16
NW = NC * NS
B_PER_W = N // NW
CHUNK = 256
SUB = CHUNK // 128
N_CHUNKS = B_PER_W // CHUNK
NBUF = 2
GROUPS = B_PER_W // 128

_SC_PARAMS = pltpu.CompilerParams(
    use_tc_tiling_on_sc=False, needs_layout_passes=False)

_BLK = 2000
_N_A = CUT1 // _BLK
_N_B = (CUT2 - CUT1) // _BLK


def _preproject_body(emb0_ref, emb1t_ref, w0_ref, w1_ref, out_ref):
    i = pl.program_id(0)

    @pl.when(i < _N_A)
    def _():
        out_ref[...] = jnp.dot(emb0_ref[...], w0_ref[...],
                               preferred_element_type=jnp.float32)

    @pl.when(i >= _N_A)
    def _():
        out_ref[...] = jnp.dot(emb1t_ref[...], w1_ref[...],
                               preferred_element_type=jnp.float32)


def _preproject(emb_0, emb_1, proj_0, proj_1):
    w0 = proj_0.T * SCALE
    w1 = proj_1.T * SCALE
    return pl.pallas_call(
        _preproject_body,
        grid=(_N_A + _N_B,),
        in_specs=[
            pl.BlockSpec((_BLK, 128), lambda i: (jnp.minimum(i, _N_A - 1), 0)),
            pl.BlockSpec((_BLK, 32), lambda i: (jnp.clip(i - _N_A, 0, _N_B - 1), 0)),
            pl.BlockSpec((128, 128), lambda i: (0, 0)),
            pl.BlockSpec((32, D_PROJ), lambda i: (0, 0)),
        ],
        out_specs=pl.BlockSpec((_BLK, D_PROJ), lambda i: (i, 0)),
        out_shape=jax.ShapeDtypeStruct((CUT2, D_PROJ), jnp.float32),
    )(emb_0, emb_1, w0, w1)


_TPW = (N_TOKEN - CUT2) // NW
_CHP = 5000
_NCHP = _TPW // _CHP


def _sc_pack_body(emb2t_hbm, emb2d_hbm, in_v, out_v):
    wid = lax.axis_index("s") * NC + lax.axis_index("c")
    lane_iota = lax.iota(jnp.int32, 16)

    def chunk(c, carry):
        tok0 = wid * _TPW + c * _CHP
        pltpu.sync_copy(emb2t_hbm.at[:, pl.ds(tok0, _CHP)], in_v)

        def group(g, carry2):
            rows = g * 16 + lane_iota
            for j in range(8):
                vals = in_v[j, pl.ds(g * 16, 16)]
                plsc.store_scatter(
                    out_v, [rows, jnp.full((16,), j, jnp.int32)], vals)
            return carry2

        lax.fori_loop(0, _CHP // 16, group, 0)
        pltpu.sync_copy(out_v, emb2d_hbm.at[pl.ds(tok0, _CHP)])
        return carry

    lax.fori_loop(0, _NCHP, chunk, 0)


def _sc_pack(emb2t):
    mesh = plsc.VectorSubcoreMesh(core_axis_name="c", subcore_axis_name="s")
    f = functools.partial(
        pl.kernel,
        mesh=mesh,
        out_type=[jax.ShapeDtypeStruct((N_TOKEN - CUT2, 8), jnp.float32)],
        scratch_types=[
            pltpu.VMEM((8, _CHP), jnp.float32),
            pltpu.VMEM((_CHP, 8), jnp.float32),
        ],
        compiler_params=_SC_PARAMS,
    )(_sc_pack_body)
    return f(emb2t)[0]


def _sc_a_body(inp_hbm, idx2_hbm, emb2_hbm,
               rows2_hbm, idxc_hbm, posc_hbm, cnt_hbm,
               inp_v, idx2_v, idxc_v, posc_v, cnt_v, rows2_v, gsem, wsem):
    wid = lax.axis_index("s") * NC + lax.axis_index("c")
    base = wid * B_PER_W
    base_row = wid * GROUPS

    pltpu.sync_copy(inp_hbm.at[pl.ds(base_row, GROUPS)], inp_v)
    pltpu.sync_copy(idx2_hbm.at[pl.ds(base_row, GROUPS)], idx2_v)

    def fire2(c, b):
        for j in range(SUB):
            pltpu.async_copy(
                emb2_hbm.at[idx2_v.at[c * SUB + j]],
                rows2_v.at[b, pl.ds(j * 128, 128)], gsem)

    def wait2(c, b):
        for j in range(SUB):
            pltpu.make_async_copy(
                emb2_hbm.at[idx2_v.at[c * SUB + j]],
                rows2_v.at[b, pl.ds(j * 128, 128)], gsem).wait()

    def firewb(c, b):
        pltpu.async_copy(rows2_v.at[b],
                         rows2_hbm.at[pl.ds(base + c * CHUNK, CHUNK)],
                         wsem.at[b])

    def waitwb(c, b):
        pltpu.make_async_copy(rows2_v.at[b],
                              rows2_hbm.at[pl.ds(base + c * CHUNK, CHUNK)],
                              wsem.at[b]).wait()

    fire2(0, 0)

    def ring_body(g, carry):
        for b in range(NBUF):
            c = g * NBUF + b
            nb = (b + 1) % NBUF

            @pl.when(c + 1 < N_CHUNKS)
            def _():
                @pl.when(c + 1 >= NBUF)
                def _():
                    waitwb(c + 1 - NBUF, nb)
                fire2(c + 1, nb)

            wait2(c, b)
            firewb(c, b)
        return carry

    lax.fori_loop(0, N_CHUNKS // NBUF, ring_body, 0)

    lane_iota = lax.iota(jnp.int32, 16)

    def scan_body(i, off):
        row = i >> 3
        lane = (i & 7) * 16
        vals = inp_v[row, pl.ds(lane, 16)]
        m = vals < CUT2
        mi = jnp.where(m, 1, 0)
        slots = off + plsc.cumsum(mi) - mi
        pos = base + i * 16 + lane_iota
        plsc.store_scatter(idxc_v, [slots >> 7, slots & 127], vals, mask=m)
        plsc.store_scatter(posc_v, [slots >> 7, slots & 127], pos, mask=m)
        return off + plsc.all_reduce_population_count(m)

    n01v = lax.fori_loop(0, GROUPS * 8, scan_body,
                         jnp.zeros((16,), jnp.int32))
    n01 = jnp.max(n01v)
    ngroups = (n01 + 127) >> 7

    @pl.when(n01 > 0)
    def _():
        lr = jnp.full((16,), (n01 - 1) >> 7, jnp.int32)
        lc = jnp.full((16,), (n01 - 1) & 127, jnp.int32)
        last_i = plsc.load_gather(idxc_v, [lr, lc])
        last_p = plsc.load_gather(posc_v, [lr, lc])
        for g in range(8):
            slots = (ngroups - 1) * 128 + g * 16 + lane_iota
            mpad = slots >= n01
            plsc.store_scatter(idxc_v, [slots >> 7, slots & 127], last_i,
                               mask=mpad)
            plsc.store_scatter(posc_v, [slots >> 7, slots & 127], last_p,
                               mask=mpad)

    cnt_v[0, pl.ds(0, 16)] = n01v
    pltpu.sync_copy(idxc_v, idxc_hbm.at[pl.ds(base_row, GROUPS)])
    pltpu.sync_copy(posc_v, posc_hbm.at[pl.ds(base_row, GROUPS)])
    pltpu.sync_copy(cnt_v, cnt_hbm.at[pl.ds(wid, 1)])
    for b in range(NBUF):
        waitwb(N_CHUNKS - NBUF + b, b)


def _sc_a(inp2d, idx2, emb_2):
    mesh = plsc.VectorSubcoreMesh(core_axis_name="c", subcore_axis_name="s")
    f = functools.partial(
        pl.kernel,
        mesh=mesh,
        out_type=[
            jax.ShapeDtypeStruct((N, 8), jnp.float32),
            jax.ShapeDtypeStruct((NW * GROUPS, 128), jnp.int32),
            jax.ShapeDtypeStruct((NW * GROUPS, 128), jnp.int32),
            jax.ShapeDtypeStruct((NW, 128), jnp.int32),
        ],
        scratch_types=[
            pltpu.VMEM((GROUPS, 128), jnp.int32),
            pltpu.VMEM((GROUPS, 128), jnp.int32),
            pltpu.VMEM((GROUPS, 128), jnp.int32),
            pltpu.VMEM((GROUPS, 128), jnp.int32),
            pltpu.VMEM((1, 128), jnp.int32),
            pltpu.VMEM((NBUF, CHUNK, 8), jnp.float32),
            pltpu.SemaphoreType.DMA,
            pltpu.SemaphoreType.DMA((NBUF,)),
        ],
        compiler_params=_SC_PARAMS,
    )(_sc_a_body)
    return f(inp2d, idx2, emb_2)


_CBLK = 4096
_N_CB = N // _CBLK


def _combine_body(rows2p_ref, w2big_ref, out_ref):
    c = jnp.dot(rows2p_ref[...], w2big_ref[...],
                preferred_element_type=jnp.float32)
    out_ref[...] = c.reshape(_CBLK, D_PROJ)


def _combine(rows2p, w2big):
    return pl.pallas_call(
        _combine_body,
        grid=(_N_CB,),
        in_specs=[
            pl.BlockSpec((_CBLK // 16, 128), lambda i: (i, 0)),
            pl.BlockSpec((128, 16 * D_PROJ), lambda i: (0, 0)),
        ],
        out_specs=pl.BlockSpec((_CBLK, D_PROJ), lambda i: (i, 0)),
        out_shape=jax.ShapeDtypeStruct((N, D_PROJ), jnp.float32),
    )(rows2p, w2big)


def _sc_b_body(pt_hbm, idxc_hbm, posc_hbm, cnt_hbm, out_hbm,
               idxc_v, posc_v, cnt_v, rows_v, psem, ssem):
    wid = lax.axis_index("s") * NC + lax.axis_index("c")
    base_row = wid * GROUPS

    pltpu.sync_copy(cnt_hbm.at[pl.ds(wid, 1)], cnt_v)
    pltpu.sync_copy(idxc_hbm.at[pl.ds(base_row, GROUPS)], idxc_v)
    pltpu.sync_copy(posc_hbm.at[pl.ds(base_row, GROUPS)], posc_v)
    n01 = jnp.max(cnt_v[0, pl.ds(0, 16)])
    ngroups = (n01 + 127) >> 7

    def fire_g(j, b):
        pltpu.async_copy(pt_hbm.at[idxc_v.at[j]], rows_v.at[b], psem.at[b])

    def wait_g(j, b):
        pltpu.make_async_copy(pt_hbm.at[idxc_v.at[j]], rows_v.at[b],
                              psem.at[b]).wait()

    def fire_s(j, b):
        pltpu.async_copy(rows_v.at[b], out_hbm.at[posc_v.at[j]], ssem.at[b])

    def wait_s(j, b):
        pltpu.make_async_copy(rows_v.at[b], out_hbm.at[posc_v.at[j]],
                              ssem.at[b]).wait()

    @pl.when(n01 > 0)
    def _():
        fire_g(0, 0)

        def pbody(g, carry):
            for b in range(2):
                j = g * 2 + b

                @pl.when(j < ngroups)
                def _():
                    @pl.when(j + 1 < ngroups)
                    def _():
                        @pl.when(j + 1 >= 2)
                        def _():
                            wait_s(j - 1, 1 - b)
                        fire_g(j + 1, 1 - b)

                    wait_g(j, b)
                    fire_s(j, b)
            return carry

        lax.fori_loop(0, (ngroups + 1) >> 1, pbody, 0)

        @pl.when(ngroups >= 1)
        def _():
            wait_s(0, 0)

        @pl.when(ngroups >= 2)
        def _():
            wait_s(0, 1)


def _sc_patch(pt, idxc, posc, cnt, out_ref):
    mesh = plsc.VectorSubcoreMesh(core_axis_name="c", subcore_axis_name="s")
    f = functools.partial(
        pl.kernel,
        mesh=mesh,
        out_type=[],
        scratch_types=[
            pltpu.VMEM((GROUPS, 128), jnp.int32),
            pltpu.VMEM((GROUPS, 128), jnp.int32),
            pltpu.VMEM((1, 128), jnp.int32),
            pltpu.VMEM((2, 128, D_PROJ), jnp.float32),
            pltpu.SemaphoreType.DMA((2,)),
            pltpu.SemaphoreType.DMA((2,)),
        ],
        compiler_params=_SC_PARAMS,
    )(_sc_b_body)
    f(pt, idxc, posc, cnt, out_ref)


def kernel(inp, emb_0, emb_1, emb_2, proj_0, proj_1, proj_2):
    inp_flat = inp.T.reshape(-1).astype(jnp.int32)
    is2 = inp_flat >= CUT2
    idx2 = jnp.where(is2, inp_flat - CUT2, inp_flat * 4)
    inp2d = inp_flat.reshape(N // 128, 128)

    pt = _preproject(emb_0, emb_1, proj_0, proj_1)
    emb2d = _sc_pack(emb_2.T)
    rows2, idxc, posc, cnt = _sc_a(inp2d, idx2.reshape(N // 128, 128), emb2d)

    w2 = proj_2.T * SCALE
    w2big = jnp.zeros((128, 16 * D_PROJ), jnp.float32)
    for t in range(16):
        w2big = w2big.at[8 * t:8 * (t + 1), 128 * t:128 * (t + 1)].set(w2)
    c2 = _combine(rows2.reshape(N // 16, 128), w2big)

    ref = jax.new_ref(c2)
    _sc_patch(pt, idxc, posc, cnt, ref)
    out = ref[...]
    return out.reshape(inp.shape[1], inp.shape[0], D_PROJ).transpose(1, 0, 2)


# speedup vs baseline: 1.3940x; 1.3940x over previous
import functools

import jax
import jax.numpy as jnp
from jax import lax
from jax.experimental import pallas as pl
from jax.experimental.pallas import tpu as pltpu
from jax.experimental.pallas import tpu_sc as plsc

N_TOKEN = 1000000
D_PROJ = 128
CUT1 = 20000
CUT2 = 200000
SCALE = float(D_PROJ) ** 0.5

N = 16384 * 20
NC, NS = 2, 16
NW = NC * NS
B_PER_W = N // NW
CHUNK = 256
SUB = CHUNK // 128
N_CHUNKS = B_PER_W // CHUNK
NBUF = 2
GROUPS = B_PER_W // 128

_SC_PARAMS = pltpu.CompilerParams(
    use_tc_tiling_on_sc=False, needs_layout_passes=False)

_BLK = 4000
_N_A = CUT1 // _BLK
_N_B = (CUT2 - CUT1) // _BLK


def _preproject_body(emb0_ref, emb1t_ref, w0_ref, w1_ref, out_ref):
    i = pl.program_id(0)

    @pl.when(i < _N_A)
    def _():
        out_ref[...] = jnp.dot(emb0_ref[...], w0_ref[...],
                               preferred_element_type=jnp.float32)

    @pl.when(i >= _N_A)
    def _():
        out_ref[...] = jnp.dot(emb1t_ref[...], w1_ref[...],
                               preferred_element_type=jnp.float32)


def _preproject(emb_0, emb_1, proj_0, proj_1):
    w0 = proj_0.T * SCALE
    w1 = proj_1.T * SCALE
    return pl.pallas_call(
        _preproject_body,
        grid=(_N_A + _N_B,),
        in_specs=[
            pl.BlockSpec((_BLK, 128), lambda i: (jnp.minimum(i, _N_A - 1), 0)),
            pl.BlockSpec((_BLK, 32), lambda i: (jnp.clip(i - _N_A, 0, _N_B - 1), 0)),
            pl.BlockSpec((128, 128), lambda i: (0, 0)),
            pl.BlockSpec((32, D_PROJ), lambda i: (0, 0)),
        ],
        out_specs=pl.BlockSpec((_BLK, D_PROJ), lambda i: (i, 0)),
        out_shape=jax.ShapeDtypeStruct((CUT2, D_PROJ), jnp.float32),
    )(emb_0, emb_1, w0, w1)


def _sc_a_body(inp_hbm, idx2_hbm, emb2_hbm,
               rows2_hbm, idxc_hbm, posc_hbm, cnt_hbm,
               inp_v, idx2_v, idxc_v, posc_v, cnt_v, rows2_v, gsem, wsem):
    wid = lax.axis_index("s") * NC + lax.axis_index("c")
    base = wid * B_PER_W
    base_row = wid * GROUPS

    pltpu.sync_copy(inp_hbm.at[pl.ds(base_row, GROUPS)], inp_v)
    pltpu.sync_copy(idx2_hbm.at[pl.ds(base_row, GROUPS)], idx2_v)

    def fire2(c, b):
        for j in range(SUB):
            pltpu.async_copy(
                emb2_hbm.at[idx2_v.at[c * SUB + j]],
                rows2_v.at[b, pl.ds(j * 128, 128)], gsem)

    def wait2(c, b):
        for j in range(SUB):
            pltpu.make_async_copy(
                emb2_hbm.at[idx2_v.at[c * SUB + j]],
                rows2_v.at[b, pl.ds(j * 128, 128)], gsem).wait()

    def firewb(c, b):
        pltpu.async_copy(rows2_v.at[b],
                         rows2_hbm.at[pl.ds(base + c * CHUNK, CHUNK)],
                         wsem.at[b])

    def waitwb(c, b):
        pltpu.make_async_copy(rows2_v.at[b],
                              rows2_hbm.at[pl.ds(base + c * CHUNK, CHUNK)],
                              wsem.at[b]).wait()

    fire2(0, 0)

    def ring_body(g, carry):
        for b in range(NBUF):
            c = g * NBUF + b
            nb = (b + 1) % NBUF

            @pl.when(c + 1 < N_CHUNKS)
            def _():
                @pl.when(c + 1 >= NBUF)
                def _():
                    waitwb(c + 1 - NBUF, nb)
                fire2(c + 1, nb)

            wait2(c, b)
            firewb(c, b)
        return carry

    lax.fori_loop(0, N_CHUNKS // NBUF, ring_body, 0)

    lane_iota = lax.iota(jnp.int32, 16)

    def scan_body(i, off):
        row = i >> 3
        lane = (i & 7) * 16
        vals = inp_v[row, pl.ds(lane, 16)]
        m = vals < CUT2
        mi = jnp.where(m, 1, 0)
        slots = off + plsc.cumsum(mi) - mi
        pos = base + i * 16 + lane_iota
        plsc.store_scatter(idxc_v, [slots >> 7, slots & 127], vals, mask=m)
        plsc.store_scatter(posc_v, [slots >> 7, slots & 127], pos, mask=m)
        return off + plsc.all_reduce_population_count(m)

    n01v = lax.fori_loop(0, GROUPS * 8, scan_body,
                         jnp.zeros((16,), jnp.int32))
    n01 = jnp.max(n01v)
    ngroups = (n01 + 127) >> 7

    @pl.when(n01 > 0)
    def _():
        lr = jnp.full((16,), (n01 - 1) >> 7, jnp.int32)
        lc = jnp.full((16,), (n01 - 1) & 127, jnp.int32)
        last_i = plsc.load_gather(idxc_v, [lr, lc])
        last_p = plsc.load_gather(posc_v, [lr, lc])
        for g in range(8):
            slots = (ngroups - 1) * 128 + g * 16 + lane_iota
            mpad = slots >= n01
            plsc.store_scatter(idxc_v, [slots >> 7, slots & 127], last_i,
                               mask=mpad)
            plsc.store_scatter(posc_v, [slots >> 7, slots & 127], last_p,
                               mask=mpad)

    cnt_v[0, pl.ds(0, 16)] = n01v
    pltpu.sync_copy(idxc_v, idxc_hbm.at[pl.ds(base_row, GROUPS)])
    pltpu.sync_copy(posc_v, posc_hbm.at[pl.ds(base_row, GROUPS)])
    pltpu.sync_copy(cnt_v, cnt_hbm.at[pl.ds(wid, 1)])
    for b in range(NBUF):
        waitwb(N_CHUNKS - NBUF + b, b)


def _sc_a(inp2d, idx2, emb_2):
    mesh = plsc.VectorSubcoreMesh(core_axis_name="c", subcore_axis_name="s")
    f = functools.partial(
        pl.kernel,
        mesh=mesh,
        out_type=[
            jax.ShapeDtypeStruct((N, 8), jnp.float32),
            jax.ShapeDtypeStruct((NW * GROUPS, 128), jnp.int32),
            jax.ShapeDtypeStruct((NW * GROUPS, 128), jnp.int32),
            jax.ShapeDtypeStruct((NW, 128), jnp.int32),
        ],
        scratch_types=[
            pltpu.VMEM((GROUPS, 128), jnp.int32),
            pltpu.VMEM((GROUPS, 128), jnp.int32),
            pltpu.VMEM((GROUPS, 128), jnp.int32),
            pltpu.VMEM((GROUPS, 128), jnp.int32),
            pltpu.VMEM((1, 128), jnp.int32),
            pltpu.VMEM((NBUF, CHUNK, 8), jnp.float32),
            pltpu.SemaphoreType.DMA,
            pltpu.SemaphoreType.DMA((NBUF,)),
        ],
        compiler_params=_SC_PARAMS,
    )(_sc_a_body)
    return f(inp2d, idx2, emb_2)


_CBLK = 8192
_N_CB = N // _CBLK


def _combine_body(rows2p_ref, w2big_ref, out_ref):
    c = jnp.dot(rows2p_ref[...], w2big_ref[...],
                preferred_element_type=jnp.float32)
    out_ref[...] = c.reshape(_CBLK, D_PROJ)


def _combine(rows2p, w2big):
    return pl.pallas_call(
        _combine_body,
        grid=(_N_CB,),
        in_specs=[
            pl.BlockSpec((_CBLK // 16, 128), lambda i: (i, 0)),
            pl.BlockSpec((128, 16 * D_PROJ), lambda i: (0, 0)),
        ],
        out_specs=pl.BlockSpec((_CBLK, D_PROJ), lambda i: (i, 0)),
        out_shape=jax.ShapeDtypeStruct((N, D_PROJ), jnp.float32),
    )(rows2p, w2big)


def _sc_b_body(pt_hbm, idxc_hbm, posc_hbm, cnt_hbm, out_hbm,
               idxc_v, posc_v, cnt_v, rows_v, psem, ssem):
    wid = lax.axis_index("s") * NC + lax.axis_index("c")
    base_row = wid * GROUPS

    pltpu.sync_copy(cnt_hbm.at[pl.ds(wid, 1)], cnt_v)
    pltpu.sync_copy(idxc_hbm.at[pl.ds(base_row, GROUPS)], idxc_v)
    pltpu.sync_copy(posc_hbm.at[pl.ds(base_row, GROUPS)], posc_v)
    n01 = jnp.max(cnt_v[0, pl.ds(0, 16)])
    ngroups = (n01 + 127) >> 7

    def fire_g(j, b):
        pltpu.async_copy(pt_hbm.at[idxc_v.at[j]], rows_v.at[b], psem.at[b])

    def wait_g(j, b):
        pltpu.make_async_copy(pt_hbm.at[idxc_v.at[j]], rows_v.at[b],
                              psem.at[b]).wait()

    def fire_s(j, b):
        pltpu.async_copy(rows_v.at[b], out_hbm.at[posc_v.at[j]], ssem.at[b])

    def wait_s(j, b):
        pltpu.make_async_copy(rows_v.at[b], out_hbm.at[posc_v.at[j]],
                              ssem.at[b]).wait()

    @pl.when(n01 > 0)
    def _():
        fire_g(0, 0)

        def pbody(g, carry):
            for b in range(2):
                j = g * 2 + b

                @pl.when(j < ngroups)
                def _():
                    @pl.when(j + 1 < ngroups)
                    def _():
                        @pl.when(j + 1 >= 2)
                        def _():
                            wait_s(j - 1, 1 - b)
                        fire_g(j + 1, 1 - b)

                    wait_g(j, b)
                    fire_s(j, b)
            return carry

        lax.fori_loop(0, (ngroups + 1) >> 1, pbody, 0)

        @pl.when(ngroups >= 1)
        def _():
            wait_s(0, 0)

        @pl.when(ngroups >= 2)
        def _():
            wait_s(0, 1)


def _sc_patch(pt, idxc, posc, cnt, out_ref):
    mesh = plsc.VectorSubcoreMesh(core_axis_name="c", subcore_axis_name="s")
    f = functools.partial(
        pl.kernel,
        mesh=mesh,
        out_type=[],
        scratch_types=[
            pltpu.VMEM((GROUPS, 128), jnp.int32),
            pltpu.VMEM((GROUPS, 128), jnp.int32),
            pltpu.VMEM((1, 128), jnp.int32),
            pltpu.VMEM((2, 128, D_PROJ), jnp.float32),
            pltpu.SemaphoreType.DMA((2,)),
            pltpu.SemaphoreType.DMA((2,)),
        ],
        compiler_params=_SC_PARAMS,
    )(_sc_b_body)
    f(pt, idxc, posc, cnt, out_ref)


def kernel(inp, emb_0, emb_1, emb_2, proj_0, proj_1, proj_2):
    inp_flat = inp.T.reshape(-1).astype(jnp.int32)
    is2 = inp_flat >= CUT2
    idx2 = jnp.where(is2, inp_flat - CUT2, inp_flat * 4)
    inp2d = inp_flat.reshape(N // 128, 128)

    pt = _preproject(emb_0, emb_1, proj_0, proj_1)
    rows2, idxc, posc, cnt = _sc_a(inp2d, idx2.reshape(N // 128, 128), emb_2)

    w2 = proj_2.T * SCALE
    w2big = jnp.zeros((128, 16 * D_PROJ), jnp.float32)
    for t in range(16):
        w2big = w2big.at[8 * t:8 * (t + 1), 128 * t:128 * (t + 1)].set(w2)
    c2 = _combine(rows2.reshape(N // 16, 128), w2big)

    ref = jax.new_ref(c2)
    _sc_patch(pt, idxc, posc, cnt, ref)
    out = ref[...]
    return out.reshape(inp.shape[1], inp.shape[0], D_PROJ).transpose(1, 0, 2)


# speedup vs baseline: 1.4510x; 1.0409x over previous
import functools

import jax
import jax.numpy as jnp
from jax import lax
from jax.experimental import pallas as pl
from jax.experimental.pallas import tpu as pltpu
from jax.experimental.pallas import tpu_sc as plsc

N_TOKEN = 1000000
D_PROJ = 128
CUT1 = 20000
CUT2 = 200000
SCALE = float(D_PROJ) ** 0.5

N = 16384 * 20
NC, NS = 2, 16
NW = NC * NS
B_PER_W = N // NW
CHUNK = 256
SUB = CHUNK // 128
N_CHUNKS = B_PER_W // CHUNK
NBUF = 2
GROUPS = B_PER_W // 128

_SC_PARAMS = pltpu.CompilerParams(
    use_tc_tiling_on_sc=False, needs_layout_passes=False)

_BLK = 10000
_N_A = CUT1 // _BLK
_N_B = (CUT2 - CUT1) // _BLK


def _preproject_body(emb0_ref, emb1t_ref, w0_ref, w1_ref, out_ref):
    i = pl.program_id(0)

    @pl.when(i < _N_A)
    def _():
        out_ref[...] = jnp.dot(emb0_ref[...], w0_ref[...],
                               preferred_element_type=jnp.float32)

    @pl.when(i >= _N_A)
    def _():
        out_ref[...] = jnp.dot(emb1t_ref[...], w1_ref[...],
                               preferred_element_type=jnp.float32)


def _preproject(emb_0, emb_1, proj_0, proj_1):
    w0 = proj_0.T * SCALE
    w1 = proj_1.T * SCALE
    return pl.pallas_call(
        _preproject_body,
        grid=(_N_A + _N_B,),
        in_specs=[
            pl.BlockSpec((_BLK, 128), lambda i: (jnp.minimum(i, _N_A - 1), 0)),
            pl.BlockSpec((_BLK, 32), lambda i: (jnp.clip(i - _N_A, 0, _N_B - 1), 0)),
            pl.BlockSpec((128, 128), lambda i: (0, 0)),
            pl.BlockSpec((32, D_PROJ), lambda i: (0, 0)),
        ],
        out_specs=pl.BlockSpec((_BLK, D_PROJ), lambda i: (i, 0)),
        out_shape=jax.ShapeDtypeStruct((CUT2, D_PROJ), jnp.float32),
    )(emb_0, emb_1, w0, w1)


def _sc_a_body(inp_hbm, idx2_hbm, emb2_hbm,
               rows2_hbm, idxc_hbm, posc_hbm, cnt_hbm,
               inp_v, idx2_v, idxc_v, posc_v, cnt_v, rows2_v, gsem, wsem):
    wid = lax.axis_index("s") * NC + lax.axis_index("c")
    base = wid * B_PER_W
    base_row = wid * GROUPS

    pltpu.sync_copy(inp_hbm.at[pl.ds(base_row, GROUPS)], inp_v)
    pltpu.sync_copy(idx2_hbm.at[pl.ds(base_row, GROUPS)], idx2_v)

    def fire2(c, b):
        for j in range(SUB):
            pltpu.async_copy(
                emb2_hbm.at[idx2_v.at[c * SUB + j]],
                rows2_v.at[b, pl.ds(j * 128, 128)], gsem)

    def wait2(c, b):
        for j in range(SUB):
            pltpu.make_async_copy(
                emb2_hbm.at[idx2_v.at[c * SUB + j]],
                rows2_v.at[b, pl.ds(j * 128, 128)], gsem).wait()

    def firewb(c, b):
        pltpu.async_copy(rows2_v.at[b],
                         rows2_hbm.at[pl.ds(base + c * CHUNK, CHUNK)],
                         wsem.at[b])

    def waitwb(c, b):
        pltpu.make_async_copy(rows2_v.at[b],
                              rows2_hbm.at[pl.ds(base + c * CHUNK, CHUNK)],
                              wsem.at[b]).wait()

    fire2(0, 0)

    def ring_body(g, carry):
        for b in range(NBUF):
            c = g * NBUF + b
            nb = (b + 1) % NBUF

            @pl.when(c + 1 < N_CHUNKS)
            def _():
                @pl.when(c + 1 >= NBUF)
                def _():
                    waitwb(c + 1 - NBUF, nb)
                fire2(c + 1, nb)

            wait2(c, b)
            firewb(c, b)
        return carry

    lax.fori_loop(0, N_CHUNKS // NBUF, ring_body, 0)

    lane_iota = lax.iota(jnp.int32, 16)

    def scan_body(i, off):
        row = i >> 3
        lane = (i & 7) * 16
        vals = inp_v[row, pl.ds(lane, 16)]
        m = vals < CUT2
        mi = jnp.where(m, 1, 0)
        slots = off + plsc.cumsum(mi) - mi
        pos = base + i * 16 + lane_iota
        plsc.store_scatter(idxc_v, [slots >> 7, slots & 127], vals, mask=m)
        plsc.store_scatter(posc_v, [slots >> 7, slots & 127], pos, mask=m)
        return off + plsc.all_reduce_population_count(m)

    n01v = lax.fori_loop(0, GROUPS * 8, scan_body,
                         jnp.zeros((16,), jnp.int32))
    n01 = jnp.max(n01v)
    ngroups = (n01 + 127) >> 7

    @pl.when(n01 > 0)
    def _():
        lr = jnp.full((16,), (n01 - 1) >> 7, jnp.int32)
        lc = jnp.full((16,), (n01 - 1) & 127, jnp.int32)
        last_i = plsc.load_gather(idxc_v, [lr, lc])
        last_p = plsc.load_gather(posc_v, [lr, lc])
        for g in range(8):
            slots = (ngroups - 1) * 128 + g * 16 + lane_iota
            mpad = slots >= n01
            plsc.store_scatter(idxc_v, [slots >> 7, slots & 127], last_i,
                               mask=mpad)
            plsc.store_scatter(posc_v, [slots >> 7, slots & 127], last_p,
                               mask=mpad)

    cnt_v[0, pl.ds(0, 16)] = n01v
    pltpu.sync_copy(idxc_v, idxc_hbm.at[pl.ds(base_row, GROUPS)])
    pltpu.sync_copy(posc_v, posc_hbm.at[pl.ds(base_row, GROUPS)])
    pltpu.sync_copy(cnt_v, cnt_hbm.at[pl.ds(wid, 1)])
    for b in range(NBUF):
        waitwb(N_CHUNKS - NBUF + b, b)


def _sc_a(inp2d, idx2, emb_2):
    mesh = plsc.VectorSubcoreMesh(core_axis_name="c", subcore_axis_name="s")
    f = functools.partial(
        pl.kernel,
        mesh=mesh,
        out_type=[
            jax.ShapeDtypeStruct((N, 8), jnp.float32),
            jax.ShapeDtypeStruct((NW * GROUPS, 128), jnp.int32),
            jax.ShapeDtypeStruct((NW * GROUPS, 128), jnp.int32),
            jax.ShapeDtypeStruct((NW, 128), jnp.int32),
        ],
        scratch_types=[
            pltpu.VMEM((GROUPS, 128), jnp.int32),
            pltpu.VMEM((GROUPS, 128), jnp.int32),
            pltpu.VMEM((GROUPS, 128), jnp.int32),
            pltpu.VMEM((GROUPS, 128), jnp.int32),
            pltpu.VMEM((1, 128), jnp.int32),
            pltpu.VMEM((NBUF, CHUNK, 8), jnp.float32),
            pltpu.SemaphoreType.DMA,
            pltpu.SemaphoreType.DMA((NBUF,)),
        ],
        compiler_params=_SC_PARAMS,
    )(_sc_a_body)
    return f(inp2d, idx2, emb_2)


_CBLK = 16384
_N_CB = N // _CBLK


def _combine_body(rows2p_ref, w2big_ref, out_ref):
    c = jnp.dot(rows2p_ref[...], w2big_ref[...],
                preferred_element_type=jnp.float32)
    out_ref[...] = c.reshape(_CBLK, D_PROJ)


def _combine(rows2p, w2big):
    return pl.pallas_call(
        _combine_body,
        grid=(_N_CB,),
        in_specs=[
            pl.BlockSpec((_CBLK // 16, 128), lambda i: (i, 0)),
            pl.BlockSpec((128, 16 * D_PROJ), lambda i: (0, 0)),
        ],
        out_specs=pl.BlockSpec((_CBLK, D_PROJ), lambda i: (i, 0)),
        out_shape=jax.ShapeDtypeStruct((N, D_PROJ), jnp.float32),
    )(rows2p, w2big)


def _sc_b_body(pt_hbm, idxc_hbm, posc_hbm, cnt_hbm, out_hbm,
               idxc_v, posc_v, cnt_v, rows_v, psem, ssem):
    wid = lax.axis_index("s") * NC + lax.axis_index("c")
    base_row = wid * GROUPS

    pltpu.sync_copy(cnt_hbm.at[pl.ds(wid, 1)], cnt_v)
    pltpu.sync_copy(idxc_hbm.at[pl.ds(base_row, GROUPS)], idxc_v)
    pltpu.sync_copy(posc_hbm.at[pl.ds(base_row, GROUPS)], posc_v)
    n01 = jnp.max(cnt_v[0, pl.ds(0, 16)])
    ngroups = (n01 + 127) >> 7

    def fire_g(j, b):
        pltpu.async_copy(pt_hbm.at[idxc_v.at[j]], rows_v.at[b], psem.at[b])

    def wait_g(j, b):
        pltpu.make_async_copy(pt_hbm.at[idxc_v.at[j]], rows_v.at[b],
                              psem.at[b]).wait()

    def fire_s(j, b):
        pltpu.async_copy(rows_v.at[b], out_hbm.at[posc_v.at[j]], ssem.at[b])

    def wait_s(j, b):
        pltpu.make_async_copy(rows_v.at[b], out_hbm.at[posc_v.at[j]],
                              ssem.at[b]).wait()

    @pl.when(n01 > 0)
    def _():
        fire_g(0, 0)

        def pbody(g, carry):
            for b in range(2):
                j = g * 2 + b

                @pl.when(j < ngroups)
                def _():
                    @pl.when(j + 1 < ngroups)
                    def _():
                        @pl.when(j + 1 >= 2)
                        def _():
                            wait_s(j - 1, 1 - b)
                        fire_g(j + 1, 1 - b)

                    wait_g(j, b)
                    fire_s(j, b)
            return carry

        lax.fori_loop(0, (ngroups + 1) >> 1, pbody, 0)

        @pl.when(ngroups >= 1)
        def _():
            wait_s(0, 0)

        @pl.when(ngroups >= 2)
        def _():
            wait_s(0, 1)


def _sc_patch(pt, idxc, posc, cnt, out_ref):
    mesh = plsc.VectorSubcoreMesh(core_axis_name="c", subcore_axis_name="s")
    f = functools.partial(
        pl.kernel,
        mesh=mesh,
        out_type=[],
        scratch_types=[
            pltpu.VMEM((GROUPS, 128), jnp.int32),
            pltpu.VMEM((GROUPS, 128), jnp.int32),
            pltpu.VMEM((1, 128), jnp.int32),
            pltpu.VMEM((2, 128, D_PROJ), jnp.float32),
            pltpu.SemaphoreType.DMA((2,)),
            pltpu.SemaphoreType.DMA((2,)),
        ],
        compiler_params=_SC_PARAMS,
    )(_sc_b_body)
    f(pt, idxc, posc, cnt, out_ref)


def kernel(inp, emb_0, emb_1, emb_2, proj_0, proj_1, proj_2):
    inp_flat = inp.T.reshape(-1).astype(jnp.int32)
    is2 = inp_flat >= CUT2
    idx2 = jnp.where(is2, inp_flat - CUT2, inp_flat * 4)
    inp2d = inp_flat.reshape(N // 128, 128)

    pt = _preproject(emb_0, emb_1, proj_0, proj_1)
    rows2, idxc, posc, cnt = _sc_a(inp2d, idx2.reshape(N // 128, 128), emb_2)

    w2 = proj_2.T * SCALE
    w2big = jnp.zeros((128, 16 * D_PROJ), jnp.float32)
    for t in range(16):
        w2big = w2big.at[8 * t:8 * (t + 1), 128 * t:128 * (t + 1)].set(w2)
    c2 = _combine(rows2.reshape(N // 16, 128), w2big)

    ref = jax.new_ref(c2)
    _sc_patch(pt, idxc, posc, cnt, ref)
    out = ref[...]
    return out.reshape(inp.shape[1], inp.shape[0], D_PROJ).transpose(1, 0, 2)
